# baseline (speedup 1.0000x reference)
import functools

import jax
import jax.numpy as jnp
from jax import lax
from jax.experimental import pallas as pl
from jax.experimental.pallas import tpu as pltpu
from jax.experimental.pallas import tpu_sc as plsc

_K = 16
_N = 4096
_BM = 512
_M_SC = 1536
_M_TC = _N - _M_SC
_NW = 32
_ROWS_PER = _M_SC // _NW
_NCHUNK = _N // _K


def _dist(q, pt, sqq, sqp):
    d = lax.dot_general(
        q, pt,
        dimension_numbers=(((1,), (0,)), ((), ())),
        precision=lax.Precision.DEFAULT,
        preferred_element_type=jnp.float32,
    )
    d = -2.0 * d
    d = d + sqq
    d = d + sqp
    return d


def _select_body(q_ref, pt_ref, sqq_ref, sqp_ref, out_ref):
    d = _dist(q_ref[...], pt_ref[...], sqq_ref[...], sqp_ref[...])
    iota = lax.broadcasted_iota(jnp.int32, (1, _N), 1)
    for j in range(_K):
        im = jnp.argmin(d, axis=1).astype(jnp.int32)[:, None]
        out_ref[:, j:j + 1] = im
        d = jnp.where(iota == im, jnp.float32(jnp.inf), d)


def _dist_body(q_ref, pt_ref, sqq_ref, sqp_ref, out_ref, cm_ref):
    d = _dist(q_ref[...], pt_ref[...], sqq_ref[...], sqp_ref[...])
    out_ref[...] = d
    cm = d[:, 0:_NCHUNK]
    for j in range(1, _K):
        cm = jnp.minimum(cm, d[:, j * _NCHUNK:(j + 1) * _NCHUNK])
    cm_ref[...] = cm


def _sc_topk_body(d_hbm, cm_hbm, out_hbm, dbuf, cbuf, obuf, sem):
    del sem
    wid = lax.axis_index("s") * 2 + lax.axis_index("c")
    base = wid * _ROWS_PER
    iota16 = lax.iota(jnp.int32, _K)

    def _scalar(vec, i):
        return lax.squeeze(lax.slice(vec, (i,), (i + 1,)), (0,))

    pltpu.sync_copy(cm_hbm.at[pl.ds(base, _ROWS_PER)], cbuf)

    def row_body(r, token):
        row = base + r
        pltpu.sync_copy(d_hbm.at[row], dbuf)

        def superstep(s, carry):
            bv, bi, worst = carry
            cmv = cbuf[r, pl.ds(s * _K, _K)]
            mask = (cmv < worst) & ((s > 0) | (iota16 > 0))

            def hit_cond(st):
                mask, bv, bi, worst = st
                n = plsc.all_reduce_population_count(mask)
                return _scalar(n, 0) > 0

            def hit_body(st):
                mask, bv, bi, worst = st
                cl = _scalar(plsc.all_reduce_ffs(mask), 0)
                c = s * _K + cl
                ci = c + _NCHUNK * iota16
                x = plsc.load_gather(dbuf, [ci])
                xv, xi = plsc.sort_key_val(x, ci)
                rbv = lax.rev(bv, (0,))
                rbi = lax.rev(bi, (0,))
                take = xv < rbv
                nv = jnp.where(take, xv, rbv)
                ni = jnp.where(take, xi, rbi)
                sv, si = plsc.sort_key_val(nv, ni)
                nworst = _scalar(sv, _K - 1)
                nmask = mask & (iota16 != cl) & (cmv < nworst)
                return nmask, sv, si, nworst

            _, bv, bi, worst = lax.while_loop(
                hit_cond, hit_body, (mask, bv, bi, worst))
            return bv, bi, worst

        ci0 = _NCHUNK * iota16
        bv0, bi0 = plsc.sort_key_val(plsc.load_gather(dbuf, [ci0]), ci0)
        init = (bv0, bi0, _scalar(bv0, _K - 1))
        _, bi, _ = lax.fori_loop(0, _NCHUNK // _K, superstep, init)
        obuf[...] = bi
        pltpu.sync_copy(obuf, out_hbm.at[row])
        return token

    lax.fori_loop(0, _ROWS_PER, row_body, 0)


_sc_topk = functools.partial(
    pl.kernel,
    out_type=jax.ShapeDtypeStruct((_M_SC, _K), jnp.int32),
    mesh=plsc.VectorSubcoreMesh(core_axis_name="c", subcore_axis_name="s"),
    scratch_types=[
        pltpu.VMEM((_N,), jnp.float32),
        pltpu.VMEM((_ROWS_PER, _NCHUNK), jnp.float32),
        pltpu.VMEM((_K,), jnp.int32),
        pltpu.SemaphoreType.DMA,
    ],
    compiler_params=pltpu.CompilerParams(needs_layout_passes=False),
)(_sc_topk_body)


@jax.jit
def _knn16(v0):
    xyz = jnp.pad(v0, ((0, 0), (0, 5)))
    sq = jnp.sum(v0 ** 2, axis=-1)
    sqq = sq[:, None]
    sqp = sq[None, :]
    pt = xyz.T

    idx_tc = pl.pallas_call(
        _select_body,
        grid=(_M_TC // _BM,),
        in_specs=[
            pl.BlockSpec((_BM, 8), lambda i: (i, 0)),
            pl.BlockSpec((8, _N), lambda i: (0, 0)),
            pl.BlockSpec((_BM, 1), lambda i: (i, 0)),
            pl.BlockSpec((1, _N), lambda i: (0, 0)),
        ],
        out_specs=pl.BlockSpec((_BM, _K), lambda i: (i, 0)),
        out_shape=jax.ShapeDtypeStruct((_M_TC, _K), jnp.int32),
    )(xyz, pt, sqq, sqp)

    d_sc, cm_sc = pl.pallas_call(
        _dist_body,
        grid=(_M_SC // _BM,),
        in_specs=[
            pl.BlockSpec((_BM, 8), lambda i: (i + _M_TC // _BM, 0)),
            pl.BlockSpec((8, _N), lambda i: (0, 0)),
            pl.BlockSpec((_BM, 1), lambda i: (i + _M_TC // _BM, 0)),
            pl.BlockSpec((1, _N), lambda i: (0, 0)),
        ],
        out_specs=[
            pl.BlockSpec((_BM, _N), lambda i: (i, 0)),
            pl.BlockSpec((_BM, _NCHUNK), lambda i: (i, 0)),
        ],
        out_shape=[
            jax.ShapeDtypeStruct((_M_SC, _N), jnp.float32),
            jax.ShapeDtypeStruct((_M_SC, _NCHUNK), jnp.float32),
        ],
    )(xyz, pt, sqq, sqp)

    idx_sc = _sc_topk(d_sc, cm_sc)
    return jnp.concatenate([idx_tc, idx_sc], axis=0)


def kernel(F, vertices):
    del F
    return _knn16(vertices[0])


# speedup vs baseline: 1.1760x; 1.1760x over previous
import functools

import jax
import jax.numpy as jnp
from jax import lax
from jax.experimental import pallas as pl
from jax.experimental.pallas import tpu as pltpu
from jax.experimental.pallas import tpu_sc as plsc

_K = 16
_N = 4096
_BM = 512
_M_SC = 1536
_M_TC = _N - _M_SC
_NW = 32
_ROWS_PER = _M_SC // _NW
_NCHUNK = _N // _K


def _dist(q, pt, sqq, sqp):
    d = lax.dot_general(
        q, pt,
        dimension_numbers=(((1,), (0,)), ((), ())),
        precision=lax.Precision.DEFAULT,
        preferred_element_type=jnp.float32,
    )
    d = -2.0 * d
    d = d + sqq
    d = d + sqp
    return d


def _select_body(q_ref, pt_ref, sqq_ref, sqp_ref, out_ref):
    d = _dist(q_ref[...], pt_ref[...], sqq_ref[...], sqp_ref[...])
    iota = lax.broadcasted_iota(jnp.int32, (1, _N), 1)
    for j in range(_K):
        im = jnp.argmin(d, axis=1).astype(jnp.int32)[:, None]
        out_ref[:, j:j + 1] = im
        d = jnp.where(iota == im, jnp.float32(jnp.inf), d)


def _dist_body(q_ref, pt_ref, sqq_ref, sqp_ref, out_ref, cm_ref):
    d = _dist(q_ref[...], pt_ref[...], sqq_ref[...], sqp_ref[...])
    out_ref[...] = d
    cm = d[:, 0:_NCHUNK]
    for j in range(1, _K):
        cm = jnp.minimum(cm, d[:, j * _NCHUNK:(j + 1) * _NCHUNK])
    cm_ref[...] = cm


def _sc_topk_body(d_hbm, cm_hbm, out_hbm, dbuf0, dbuf1, cbuf, obuf, sem0, sem1):
    wid = lax.axis_index("s") * 2 + lax.axis_index("c")
    base = wid * _ROWS_PER
    iota16 = lax.iota(jnp.int32, _K)
    sems = (sem0, sem1)

    def _scalar(vec, i):
        return lax.squeeze(lax.slice(vec, (i,), (i + 1,)), (0,))

    pltpu.sync_copy(cm_hbm.at[pl.ds(base, _ROWS_PER)], cbuf)

    def _row(r, dbuf):
        def superstep(s, carry):
            bv, bi, worst = carry
            cmv = cbuf[r, pl.ds(s * _K, _K)]
            mask = (cmv < worst) & ((s > 0) | (iota16 > 0))

            def hit_cond(st):
                mask, bv, bi, worst = st
                n = plsc.all_reduce_population_count(mask)
                return _scalar(n, 0) > 0

            def hit_body(st):
                mask, bv, bi, worst = st
                cl = _scalar(plsc.all_reduce_ffs(mask), 0)
                c = s * _K + cl
                ci = c + _NCHUNK * iota16
                x = plsc.load_gather(dbuf, [ci])
                xv, xi = plsc.sort_key_val(x, ci)
                rbv = lax.rev(bv, (0,))
                rbi = lax.rev(bi, (0,))
                take = xv < rbv
                nv = jnp.where(take, xv, rbv)
                ni = jnp.where(take, xi, rbi)
                sv, si = plsc.sort_key_val(nv, ni)
                nworst = _scalar(sv, _K - 1)
                nmask = mask & (iota16 != cl) & (cmv < nworst)
                return nmask, sv, si, nworst

            _, bv, bi, worst = lax.while_loop(
                hit_cond, hit_body, (mask, bv, bi, worst))
            return bv, bi, worst

        ci0 = _NCHUNK * iota16
        bv0, bi0 = plsc.sort_key_val(plsc.load_gather(dbuf, [ci0]), ci0)
        init = (bv0, bi0, _scalar(bv0, _K - 1))
        _, bi, _ = lax.fori_loop(0, _NCHUNK // _K, superstep, init)
        obuf[r, :] = bi

    bufs = (dbuf0, dbuf1)

    def _start(row, b):
        pltpu.make_async_copy(d_hbm.at[row], bufs[b], sems[b]).start()

    def _wait(b):
        pltpu.make_async_copy(d_hbm.at[base], bufs[b], sems[b]).wait()

    _start(base, 0)

    def pair_body(p, token):
        r0 = 2 * p
        r1 = r0 + 1
        _start(base + r1, 1)
        _wait(0)
        _row(r0, dbuf0)
        rn = jnp.minimum(r0 + 2, _ROWS_PER - 1)
        _start(base + rn, 0)
        _wait(1)
        _row(r1, dbuf1)
        return token

    lax.fori_loop(0, _ROWS_PER // 2, pair_body, 0)
    _wait(0)
    pltpu.sync_copy(obuf, out_hbm.at[pl.ds(base, _ROWS_PER)])


_sc_topk = functools.partial(
    pl.kernel,
    out_type=jax.ShapeDtypeStruct((_M_SC, _K), jnp.int32),
    mesh=plsc.VectorSubcoreMesh(core_axis_name="c", subcore_axis_name="s"),
    scratch_types=[
        pltpu.VMEM((_N,), jnp.float32),
        pltpu.VMEM((_N,), jnp.float32),
        pltpu.VMEM((_ROWS_PER, _NCHUNK), jnp.float32),
        pltpu.VMEM((_ROWS_PER, _K), jnp.int32),
        pltpu.SemaphoreType.DMA,
        pltpu.SemaphoreType.DMA,
    ],
    compiler_params=pltpu.CompilerParams(needs_layout_passes=False),
)(_sc_topk_body)


@jax.jit
def _knn16(v0):
    xyz = jnp.pad(v0, ((0, 0), (0, 5)))
    sq = jnp.sum(v0 ** 2, axis=-1)
    sqq = sq[:, None]
    sqp = sq[None, :]
    pt = xyz.T

    idx_tc = pl.pallas_call(
        _select_body,
        grid=(_M_TC // _BM,),
        in_specs=[
            pl.BlockSpec((_BM, 8), lambda i: (i, 0)),
            pl.BlockSpec((8, _N), lambda i: (0, 0)),
            pl.BlockSpec((_BM, 1), lambda i: (i, 0)),
            pl.BlockSpec((1, _N), lambda i: (0, 0)),
        ],
        out_specs=pl.BlockSpec((_BM, _K), lambda i: (i, 0)),
        out_shape=jax.ShapeDtypeStruct((_M_TC, _K), jnp.int32),
    )(xyz, pt, sqq, sqp)

    d_sc, cm_sc = pl.pallas_call(
        _dist_body,
        grid=(_M_SC // _BM,),
        in_specs=[
            pl.BlockSpec((_BM, 8), lambda i: (i + _M_TC // _BM, 0)),
            pl.BlockSpec((8, _N), lambda i: (0, 0)),
            pl.BlockSpec((_BM, 1), lambda i: (i + _M_TC // _BM, 0)),
            pl.BlockSpec((1, _N), lambda i: (0, 0)),
        ],
        out_specs=[
            pl.BlockSpec((_BM, _N), lambda i: (i, 0)),
            pl.BlockSpec((_BM, _NCHUNK), lambda i: (i, 0)),
        ],
        out_shape=[
            jax.ShapeDtypeStruct((_M_SC, _N), jnp.float32),
            jax.ShapeDtypeStruct((_M_SC, _NCHUNK), jnp.float32),
        ],
    )(xyz, pt, sqq, sqp)

    idx_sc = _sc_topk(d_sc, cm_sc)
    return jnp.concatenate([idx_tc, idx_sc], axis=0)


def kernel(F, vertices):
    del F
    return _knn16(vertices[0])


# speedup vs baseline: 1.2167x; 1.0346x over previous
import functools

import jax
import jax.numpy as jnp
from jax import lax
from jax.experimental import pallas as pl
from jax.experimental.pallas import tpu as pltpu
from jax.experimental.pallas import tpu_sc as plsc

_K = 16
_N = 4096
_BM = 512
_M_SC = 1024
_M_TC = _N - _M_SC
_NW = 32
_ROWS_PER = _M_SC // _NW
_NCHUNK = _N // _K


def _dist(q, pt, sqq, sqp):
    d = lax.dot_general(
        q, pt,
        dimension_numbers=(((1,), (0,)), ((), ())),
        precision=lax.Precision.DEFAULT,
        preferred_element_type=jnp.float32,
    )
    d = -2.0 * d
    d = d + sqq
    d = d + sqp
    return d


def _select_body(q_ref, pt_ref, sqq_ref, sqp_ref, out_ref):
    d = _dist(q_ref[...], pt_ref[...], sqq_ref[...], sqp_ref[...])
    iota = lax.broadcasted_iota(jnp.int32, (1, _N), 1)
    for j in range(_K):
        im = jnp.argmin(d, axis=1).astype(jnp.int32)[:, None]
        out_ref[:, j:j + 1] = im
        d = jnp.where(iota == im, jnp.float32(jnp.inf), d)


def _dist_body(q_ref, pt_ref, sqq_ref, sqp_ref, out_ref, cm_ref):
    d = _dist(q_ref[...], pt_ref[...], sqq_ref[...], sqp_ref[...])
    out_ref[...] = d
    cm = d[:, 0:_NCHUNK]
    for j in range(1, _K):
        cm = jnp.minimum(cm, d[:, j * _NCHUNK:(j + 1) * _NCHUNK])
    cm_ref[...] = cm


def _sc_topk_body(d_hbm, cm_hbm, out_hbm, dbuf0, dbuf1, cbuf, obuf, sem0, sem1):
    wid = lax.axis_index("s") * 2 + lax.axis_index("c")
    base = wid * _ROWS_PER
    iota16 = lax.iota(jnp.int32, _K)
    sems = (sem0, sem1)

    def _scalar(vec, i):
        return lax.squeeze(lax.slice(vec, (i,), (i + 1,)), (0,))

    pltpu.sync_copy(cm_hbm.at[pl.ds(base, _ROWS_PER)], cbuf)

    def _row(r, dbuf):
        def superstep(s, carry):
            bv, bi, worst = carry
            cmv = cbuf[r, pl.ds(s * _K, _K)]
            mask = (cmv < worst) & ((s > 0) | (iota16 > 0))

            def hit_cond(st):
                mask, bv, bi, worst = st
                n = plsc.all_reduce_population_count(mask)
                return _scalar(n, 0) > 0

            def hit_body(st):
                mask, bv, bi, worst = st
                cl = _scalar(plsc.all_reduce_ffs(mask), 0)
                c = s * _K + cl
                ci = c + _NCHUNK * iota16
                x = plsc.load_gather(dbuf, [ci])
                xv, xi = plsc.sort_key_val(x, ci)
                rbv = lax.rev(bv, (0,))
                rbi = lax.rev(bi, (0,))
                take = xv < rbv
                nv = jnp.where(take, xv, rbv)
                ni = jnp.where(take, xi, rbi)
                sv, si = plsc.sort_key_val(nv, ni)
                nworst = _scalar(sv, _K - 1)
                nmask = mask & (iota16 != cl) & (cmv < nworst)
                return nmask, sv, si, nworst

            _, bv, bi, worst = lax.while_loop(
                hit_cond, hit_body, (mask, bv, bi, worst))
            return bv, bi, worst

        ci0 = _NCHUNK * iota16
        bv0, bi0 = plsc.sort_key_val(plsc.load_gather(dbuf, [ci0]), ci0)
        init = (bv0, bi0, _scalar(bv0, _K - 1))
        _, bi, _ = lax.fori_loop(0, _NCHUNK // _K, superstep, init)
        obuf[r, :] = bi

    bufs = (dbuf0, dbuf1)

    def _start(row, b):
        pltpu.make_async_copy(d_hbm.at[row], bufs[b], sems[b]).start()

    def _wait(b):
        pltpu.make_async_copy(d_hbm.at[base], bufs[b], sems[b]).wait()

    _start(base, 0)

    def pair_body(p, token):
        r0 = 2 * p
        r1 = r0 + 1
        _start(base + r1, 1)
        _wait(0)
        _row(r0, dbuf0)
        rn = jnp.minimum(r0 + 2, _ROWS_PER - 1)
        _start(base + rn, 0)
        _wait(1)
        _row(r1, dbuf1)
        return token

    lax.fori_loop(0, _ROWS_PER // 2, pair_body, 0)
    _wait(0)
    pltpu.sync_copy(obuf, out_hbm.at[pl.ds(base, _ROWS_PER)])


_sc_topk = functools.partial(
    pl.kernel,
    out_type=jax.ShapeDtypeStruct((_M_SC, _K), jnp.int32),
    mesh=plsc.VectorSubcoreMesh(core_axis_name="c", subcore_axis_name="s"),
    scratch_types=[
        pltpu.VMEM((_N,), jnp.float32),
        pltpu.VMEM((_N,), jnp.float32),
        pltpu.VMEM((_ROWS_PER, _NCHUNK), jnp.float32),
        pltpu.VMEM((_ROWS_PER, _K), jnp.int32),
        pltpu.SemaphoreType.DMA,
        pltpu.SemaphoreType.DMA,
    ],
    compiler_params=pltpu.CompilerParams(needs_layout_passes=False),
)(_sc_topk_body)


@jax.jit
def _knn16(v0):
    xyz = jnp.pad(v0, ((0, 0), (0, 5)))
    sq = jnp.sum(v0 ** 2, axis=-1)
    sqq = sq[:, None]
    sqp = sq[None, :]
    pt = xyz.T

    idx_tc = pl.pallas_call(
        _select_body,
        grid=(_M_TC // _BM,),
        in_specs=[
            pl.BlockSpec((_BM, 8), lambda i: (i, 0)),
            pl.BlockSpec((8, _N), lambda i: (0, 0)),
            pl.BlockSpec((_BM, 1), lambda i: (i, 0)),
            pl.BlockSpec((1, _N), lambda i: (0, 0)),
        ],
        out_specs=pl.BlockSpec((_BM, _K), lambda i: (i, 0)),
        out_shape=jax.ShapeDtypeStruct((_M_TC, _K), jnp.int32),
    )(xyz, pt, sqq, sqp)

    d_sc, cm_sc = pl.pallas_call(
        _dist_body,
        grid=(_M_SC // _BM,),
        in_specs=[
            pl.BlockSpec((_BM, 8), lambda i: (i + _M_TC // _BM, 0)),
            pl.BlockSpec((8, _N), lambda i: (0, 0)),
            pl.BlockSpec((_BM, 1), lambda i: (i + _M_TC // _BM, 0)),
            pl.BlockSpec((1, _N), lambda i: (0, 0)),
        ],
        out_specs=[
            pl.BlockSpec((_BM, _N), lambda i: (i, 0)),
            pl.BlockSpec((_BM, _NCHUNK), lambda i: (i, 0)),
        ],
        out_shape=[
            jax.ShapeDtypeStruct((_M_SC, _N), jnp.float32),
            jax.ShapeDtypeStruct((_M_SC, _NCHUNK), jnp.float32),
        ],
    )(xyz, pt, sqq, sqp)

    idx_sc = _sc_topk(d_sc, cm_sc)
    return jnp.concatenate([idx_tc, idx_sc], axis=0)


def kernel(F, vertices):
    del F
    return _knn16(vertices[0])
